# baseline (speedup 1.0000x reference)
import functools
import jax
import jax.numpy as jnp
from jax import lax
from jax.experimental import pallas as pl
from jax.experimental.pallas import tpu as pltpu
from jax.experimental.pallas import tpu_sc as plsc

_NC = 2
_NS = 16
_NW = _NC * _NS
_MP = 20480
_APW = _MP // _NW
_NP = 208


def _sc_body(g0_h, g1_h, g2_h, g3_h, mas_h,
             a0_h, a1_h, a2_h, a3_h, u_h,
             match_h, cnt_h,
             g0v, g1v, g2v, g3v, masv,
             a0v, a1v, a2v, a3v, uv,
             matchv, cntv):
    wid = lax.axis_index("s") * _NC + lax.axis_index("c")
    base = wid * _APW
    pltpu.sync_copy(g0_h, g0v)
    pltpu.sync_copy(g1_h, g1v)
    pltpu.sync_copy(g2_h, g2v)
    pltpu.sync_copy(g3_h, g3v)
    pltpu.sync_copy(mas_h, masv)
    pltpu.sync_copy(a0_h.at[pl.ds(base, _APW)], a0v)
    pltpu.sync_copy(a1_h.at[pl.ds(base, _APW)], a1v)
    pltpu.sync_copy(a2_h.at[pl.ds(base, _APW)], a2v)
    pltpu.sync_copy(a3_h.at[pl.ds(base, _APW)], a3v)
    pltpu.sync_copy(u_h.at[pl.ds(base, _APW)], uv)
    mas = masv[pl.ds(0, 16)][0]

    def anchor_vec(a, _):
        s = a * 16
        vax0 = a0v[pl.ds(s, 16)]
        vay0 = a1v[pl.ds(s, 16)]
        vax1 = a2v[pl.ds(s, 16)]
        vay1 = a3v[pl.ds(s, 16)]
        vu = uv[pl.ds(s, 16)]
        an_size = jnp.maximum(vax1 - vax0, vay1 - vay0) - vu
        acx = (vax0 + vax1) * 0.5
        acy = (vay0 + vay1) * 0.5
        uu = vu * vu

        def gt_chunk(c, carry):
            first, cnt = carry
            t = c * 16
            vg0 = g0v[pl.ds(t, 16)]
            vg1 = g1v[pl.ds(t, 16)]
            vg2 = g2v[pl.ds(t, 16)]
            vg3 = g3v[pl.ds(t, 16)]
            for l in range(16):
                g0 = vg0[l]
                g1 = vg1[l]
                g2 = vg2[l]
                g3 = vg3[l]
                gup = jnp.maximum(jnp.maximum(g2 - g0, g3 - g1) * 2.0, mas)
                gcx = (g0 + g2) * 0.5
                gcy = (g1 + g3) * 0.5
                m = jnp.minimum(jnp.full((16,), g0, jnp.float32) - vax0,
                                jnp.full((16,), g1, jnp.float32) - vay0)
                m = jnp.minimum(m, vax1 - jnp.full((16,), g2, jnp.float32))
                m = jnp.minimum(m, vay1 - jnp.full((16,), g3, jnp.float32))
                m = jnp.minimum(m, jnp.full((16,), gup, jnp.float32) - an_size)
                dx = jnp.full((16,), gcx, jnp.float32) - acx
                dy = jnp.full((16,), gcy, jnp.float32) - acy
                m = jnp.minimum(m, uu - (dx * dx + dy * dy))
                ok = m >= 0.0
                idxv = jnp.full((16,), t + l, jnp.int32)
                first = jnp.minimum(first, jnp.where(ok, idxv, _NP))
                cnt = cnt + jnp.where(ok, 1, 0).astype(jnp.int32)
            return first, cnt

        first0 = jnp.full((16,), _NP, jnp.int32)
        cnt0 = jnp.zeros((16,), jnp.int32)
        first, cnt = lax.fori_loop(0, _NP // 16, gt_chunk, (first0, cnt0))
        matchv[pl.ds(s, 16)] = jnp.where(cnt > 0, first, 0)
        cntv[pl.ds(s, 16)] = cnt
        return _

    lax.fori_loop(0, _APW // 16, anchor_vec, 0)
    pltpu.sync_copy(matchv, match_h.at[pl.ds(base, _APW)])
    pltpu.sync_copy(cntv, cnt_h.at[pl.ds(base, _APW)])


def kernel(gt_boxes, anchor_boxes, unit_lengths, min_anchor_size):
    n = gt_boxes.shape[0]
    m = anchor_boxes.shape[0]
    g0 = jnp.pad(gt_boxes[:, 0], (0, _NP - n))
    g1 = jnp.pad(gt_boxes[:, 1], (0, _NP - n))
    g2 = jnp.pad(gt_boxes[:, 2], (0, _NP - n), constant_values=1e9)
    g3 = jnp.pad(gt_boxes[:, 3], (0, _NP - n))
    a0 = jnp.pad(anchor_boxes[:, 0], (0, _MP - m))
    a1 = jnp.pad(anchor_boxes[:, 1], (0, _MP - m))
    a2 = jnp.pad(anchor_boxes[:, 2], (0, _MP - m))
    a3 = jnp.pad(anchor_boxes[:, 3], (0, _MP - m))
    u = jnp.pad(unit_lengths, (0, _MP - m))
    mas = jnp.full((16,), jnp.asarray(min_anchor_size, jnp.float32))

    mesh = plsc.VectorSubcoreMesh(core_axis_name="c", subcore_axis_name="s")
    run = functools.partial(
        pl.kernel,
        mesh=mesh,
        out_type=[
            jax.ShapeDtypeStruct((_MP,), jnp.int32),
            jax.ShapeDtypeStruct((_MP,), jnp.int32),
        ],
        scratch_types=[
            pltpu.VMEM((_NP,), jnp.float32),
            pltpu.VMEM((_NP,), jnp.float32),
            pltpu.VMEM((_NP,), jnp.float32),
            pltpu.VMEM((_NP,), jnp.float32),
            pltpu.VMEM((16,), jnp.float32),
            pltpu.VMEM((_APW,), jnp.float32),
            pltpu.VMEM((_APW,), jnp.float32),
            pltpu.VMEM((_APW,), jnp.float32),
            pltpu.VMEM((_APW,), jnp.float32),
            pltpu.VMEM((_APW,), jnp.float32),
            pltpu.VMEM((_APW,), jnp.int32),
            pltpu.VMEM((_APW,), jnp.int32),
        ],
    )(_sc_body)
    matches_p, cnt_p = run(g0, g1, g2, g3, mas, a0, a1, a2, a3, u)
    matches = matches_p[:m]
    match_labels = (cnt_p[:m] == 1).astype(jnp.int8)
    return (matches, match_labels)


# speedup vs baseline: 2.1986x; 2.1986x over previous
import functools
import jax
import jax.numpy as jnp
from jax import lax
from jax.experimental import pallas as pl
from jax.experimental.pallas import tpu as pltpu
from jax.experimental.pallas import tpu_sc as plsc

_BLOCK = 2560
_NC = 2
_NS = 16
_NW = _NC * _NS
_SC_M = 2560
_APW = _SC_M // _NW
_NP = 208


def _tc_block(gt_ref, anc_ref, u_ref, mas_ref, match_ref, label_ref):
    gt = gt_ref[...]
    gx0 = gt[:, 0:1]
    gy0 = gt[:, 1:2]
    gx1 = gt[:, 2:3]
    gy1 = gt[:, 3:4]
    anc = anc_ref[...]
    ax0 = anc[0:1, :]
    ay0 = anc[1:2, :]
    ax1 = anc[2:3, :]
    ay1 = anc[3:4, :]
    u = u_ref[...]
    mas = mas_ref[0, 0]

    gt_upper = jnp.maximum(gx1 - gx0, gy1 - gy0) * 2.0
    gt_upper = jnp.where(gt_upper < mas, mas, gt_upper)
    gcx = (gx0 + gx1) / 2.0
    gcy = (gy0 + gy1) / 2.0
    an_size = jnp.maximum(ax1 - ax0, ay1 - ay0) - u
    acx = (ax0 + ax1) / 2.0
    acy = (ay0 + ay1) / 2.0
    uu = u * u

    m = jnp.minimum(gx0 - ax0, gy0 - ay0)
    m = jnp.minimum(m, ax1 - gx1)
    m = jnp.minimum(m, ay1 - gy1)
    m = jnp.minimum(m, gt_upper - an_size)
    dx = gcx - acx
    dy = gcy - acy
    m = jnp.minimum(m, uu - (dx * dx + dy * dy))
    assign = m >= 0.0

    n = gt.shape[0]
    iota = lax.broadcasted_iota(jnp.int32, assign.shape, 0)
    first = jnp.min(jnp.where(assign, iota, n), axis=0, keepdims=True)
    cnt = jnp.sum(jnp.where(assign, 1, 0), axis=0, keepdims=True)
    match_ref[...] = jnp.where(first == n, 0, first)
    label_ref[...] = (cnt == 1).astype(jnp.int8)


def _tc_part(gt_boxes, anc_t, u2, mas, m):
    n = gt_boxes.shape[0]
    return pl.pallas_call(
        _tc_block,
        grid=(pl.cdiv(m, _BLOCK),),
        in_specs=[
            pl.BlockSpec((n, 4), lambda j: (0, 0)),
            pl.BlockSpec((4, _BLOCK), lambda j: (0, j)),
            pl.BlockSpec((1, _BLOCK), lambda j: (0, j)),
            pl.BlockSpec((1, 1), lambda j: (0, 0)),
        ],
        out_specs=[
            pl.BlockSpec((1, _BLOCK), lambda j: (0, j)),
            pl.BlockSpec((1, _BLOCK), lambda j: (0, j)),
        ],
        out_shape=[
            jax.ShapeDtypeStruct((1, m), jnp.int32),
            jax.ShapeDtypeStruct((1, m), jnp.int8),
        ],
        compiler_params=pltpu.CompilerParams(
            dimension_semantics=("parallel",),
            allow_input_fusion=[True, True, True, True],
        ),
    )(gt_boxes, anc_t, u2, mas)


def _sc_body(g0_h, g1_h, g2_h, g3_h, mas_h,
             a0_h, a1_h, a2_h, a3_h, u_h,
             match_h, cnt_h,
             g0v, g1v, g2v, g3v, masv,
             a0v, a1v, a2v, a3v, uv,
             matchv, cntv):
    wid = lax.axis_index("s") * _NC + lax.axis_index("c")
    base = wid * _APW
    pltpu.sync_copy(g0_h, g0v)
    pltpu.sync_copy(g1_h, g1v)
    pltpu.sync_copy(g2_h, g2v)
    pltpu.sync_copy(g3_h, g3v)
    pltpu.sync_copy(mas_h, masv)
    pltpu.sync_copy(a0_h.at[pl.ds(base, _APW)], a0v)
    pltpu.sync_copy(a1_h.at[pl.ds(base, _APW)], a1v)
    pltpu.sync_copy(a2_h.at[pl.ds(base, _APW)], a2v)
    pltpu.sync_copy(a3_h.at[pl.ds(base, _APW)], a3v)
    pltpu.sync_copy(u_h.at[pl.ds(base, _APW)], uv)
    mas = masv[pl.ds(0, 16)][0]

    def anchor_vec(a, _):
        s = a * 16
        vax0 = a0v[pl.ds(s, 16)]
        vay0 = a1v[pl.ds(s, 16)]
        vax1 = a2v[pl.ds(s, 16)]
        vay1 = a3v[pl.ds(s, 16)]
        vu = uv[pl.ds(s, 16)]
        an_size = jnp.maximum(vax1 - vax0, vay1 - vay0) - vu
        acx = (vax0 + vax1) * 0.5
        acy = (vay0 + vay1) * 0.5
        uu = vu * vu

        def gt_chunk(c, carry):
            first, cnt = carry
            t = c * 16
            vg0 = g0v[pl.ds(t, 16)]
            vg1 = g1v[pl.ds(t, 16)]
            vg2 = g2v[pl.ds(t, 16)]
            vg3 = g3v[pl.ds(t, 16)]
            for l in range(16):
                g0 = vg0[l]
                g1 = vg1[l]
                g2 = vg2[l]
                g3 = vg3[l]
                gup = jnp.maximum(jnp.maximum(g2 - g0, g3 - g1) * 2.0, mas)
                gcx = (g0 + g2) * 0.5
                gcy = (g1 + g3) * 0.5
                m = jnp.minimum(jnp.full((16,), g0, jnp.float32) - vax0,
                                jnp.full((16,), g1, jnp.float32) - vay0)
                m = jnp.minimum(m, vax1 - jnp.full((16,), g2, jnp.float32))
                m = jnp.minimum(m, vay1 - jnp.full((16,), g3, jnp.float32))
                m = jnp.minimum(m, jnp.full((16,), gup, jnp.float32) - an_size)
                dx = jnp.full((16,), gcx, jnp.float32) - acx
                dy = jnp.full((16,), gcy, jnp.float32) - acy
                m = jnp.minimum(m, uu - (dx * dx + dy * dy))
                ok = m >= 0.0
                idxv = jnp.full((16,), t + l, jnp.int32)
                first = jnp.minimum(first, jnp.where(ok, idxv, _NP))
                cnt = cnt + jnp.where(ok, 1, 0).astype(jnp.int32)
            return first, cnt

        first0 = jnp.full((16,), _NP, jnp.int32)
        cnt0 = jnp.zeros((16,), jnp.int32)
        first, cnt = lax.fori_loop(0, _NP // 16, gt_chunk, (first0, cnt0))
        matchv[pl.ds(s, 16)] = jnp.where(cnt > 0, first, 0)
        cntv[pl.ds(s, 16)] = cnt
        return _

    lax.fori_loop(0, _APW // 16, anchor_vec, 0)
    pltpu.sync_copy(matchv, match_h.at[pl.ds(base, _APW)])
    pltpu.sync_copy(cntv, cnt_h.at[pl.ds(base, _APW)])


def _sc_part(gt_boxes, anchor_sl, u_sl, mas16):
    n = gt_boxes.shape[0]
    g0 = jnp.pad(gt_boxes[:, 0], (0, _NP - n))
    g1 = jnp.pad(gt_boxes[:, 1], (0, _NP - n))
    g2 = jnp.pad(gt_boxes[:, 2], (0, _NP - n), constant_values=1e9)
    g3 = jnp.pad(gt_boxes[:, 3], (0, _NP - n))
    a0 = anchor_sl[:, 0]
    a1 = anchor_sl[:, 1]
    a2 = anchor_sl[:, 2]
    a3 = anchor_sl[:, 3]

    mesh = plsc.VectorSubcoreMesh(core_axis_name="c", subcore_axis_name="s")
    run = functools.partial(
        pl.kernel,
        mesh=mesh,
        out_type=[
            jax.ShapeDtypeStruct((_SC_M,), jnp.int32),
            jax.ShapeDtypeStruct((_SC_M,), jnp.int32),
        ],
        scratch_types=[
            pltpu.VMEM((_NP,), jnp.float32),
            pltpu.VMEM((_NP,), jnp.float32),
            pltpu.VMEM((_NP,), jnp.float32),
            pltpu.VMEM((_NP,), jnp.float32),
            pltpu.VMEM((16,), jnp.float32),
            pltpu.VMEM((_APW,), jnp.float32),
            pltpu.VMEM((_APW,), jnp.float32),
            pltpu.VMEM((_APW,), jnp.float32),
            pltpu.VMEM((_APW,), jnp.float32),
            pltpu.VMEM((_APW,), jnp.float32),
            pltpu.VMEM((_APW,), jnp.int32),
            pltpu.VMEM((_APW,), jnp.int32),
        ],
    )(_sc_body)
    return run(g0, g1, g2, g3, mas16, a0, a1, a2, a3, u_sl)


def kernel(gt_boxes, anchor_boxes, unit_lengths, min_anchor_size):
    m = anchor_boxes.shape[0]
    m_tc = m - _SC_M
    mas_f = jnp.asarray(min_anchor_size, jnp.float32)

    anc_t = anchor_boxes[:m_tc].T
    u2 = unit_lengths[:m_tc].reshape(1, m_tc)
    tc_match, tc_label = _tc_part(gt_boxes, anc_t, u2, mas_f.reshape(1, 1), m_tc)

    sc_match, sc_cnt = _sc_part(
        gt_boxes, anchor_boxes[m_tc:], unit_lengths[m_tc:],
        jnp.full((16,), mas_f))

    matches = jnp.concatenate([tc_match.reshape(m_tc), sc_match])
    labels = jnp.concatenate(
        [tc_label.reshape(m_tc), (sc_cnt == 1).astype(jnp.int8)])
    return (matches, labels)
